# baseline (speedup 1.0000x reference)
import functools

import jax
import jax.numpy as jnp
from jax import lax
from jax.experimental import pallas as pl
from jax.experimental.pallas import tpu as pltpu
from jax.experimental.pallas import tpu_sc as plsc

N_NODES = 10000
FEAT = 128
NPAD = 10240
TRASH = N_NODES
E = 320000
NC, NS = 2, 16
NW = NC * NS
EDGES_PER_TILE = E // NW
CHUNK = 128
CHUNKS_PER_TILE = 80
EPAD = NW * CHUNK * CHUNKS_PER_TILE
ROWS_PER_TILE = NPAD // NS
RB = 1024
GRID = NPAD // RB

_mesh = plsc.VectorSubcoreMesh(
    core_axis_name="c", subcore_axis_name="s", num_cores=NC, num_subcores=NS)


DEGW = 128


@functools.partial(
    pl.kernel,
    mesh=_mesh,
    out_type=jax.ShapeDtypeStruct((NC, NPAD, DEGW), jnp.float32),
    scratch_types=[
        pltpu.VMEM((CHUNKS_PER_TILE, CHUNK), jnp.int32),
        pltpu.VMEM((CHUNK, DEGW), jnp.float32),
        pltpu.VMEM_SHARED((NPAD, DEGW), jnp.float32),
        pltpu.SemaphoreType.DMA,
    ],
)
def _degree_kernel(dst_hbm, ones_hbm, zeros1_hbm, deg_out, dst_v, ones_v, deg_sh,
                   sem):
    c = lax.axis_index("c")
    s = lax.axis_index("s")
    wid = s * NC + c
    rs = s * ROWS_PER_TILE
    pltpu.sync_copy(zeros1_hbm, deg_sh.at[pl.ds(rs, ROWS_PER_TILE)])
    pltpu.sync_copy(ones_hbm, ones_v)
    pltpu.sync_copy(dst_hbm.at[wid], dst_v)
    plsc.subcore_barrier()

    win = 8
    for i in range(win):
        pltpu.async_copy(ones_v, deg_sh.at[dst_v.at[i]], sem, add=True)

    def body(i, carry):
        pltpu.make_async_copy(ones_v, deg_sh.at[dst_v.at[i]], sem).wait()
        pltpu.async_copy(ones_v, deg_sh.at[dst_v.at[i + win]], sem, add=True)
        return carry

    lax.fori_loop(0, CHUNKS_PER_TILE - win, body, 0)

    def drain(i, carry):
        pltpu.make_async_copy(ones_v, deg_sh.at[dst_v.at[i]], sem).wait()
        return carry

    lax.fori_loop(CHUNKS_PER_TILE - win, CHUNKS_PER_TILE, drain, 0)
    plsc.subcore_barrier()
    pltpu.sync_copy(deg_sh.at[pl.ds(rs, ROWS_PER_TILE)],
                    deg_out.at[c, pl.ds(rs, ROWS_PER_TILE)])


NBUF = 2
NROUNDS = CHUNKS_PER_TILE // NBUF


@functools.partial(
    pl.kernel,
    mesh=_mesh,
    out_type=jax.ShapeDtypeStruct((NC, NPAD, FEAT), jnp.float32),
    scratch_types=[
        pltpu.VMEM((CHUNKS_PER_TILE, CHUNK), jnp.int32),
        pltpu.VMEM((CHUNK,), jnp.int32),
        pltpu.VMEM((CHUNK,), jnp.int32),
        pltpu.VMEM((CHUNK, FEAT), jnp.float32),
        pltpu.VMEM((CHUNK, FEAT), jnp.float32),
        pltpu.VMEM_SHARED((NPAD, FEAT), jnp.float32),
        pltpu.SemaphoreType.DMA,
        pltpu.SemaphoreType.DMA,
        pltpu.SemaphoreType.DMA,
        pltpu.SemaphoreType.DMA,
        pltpu.SemaphoreType.DMA,
        pltpu.SemaphoreType.DMA,
    ],
)
def _aggregate_kernel(hs_hbm, src_hbm, dst_hbm, zeros2_hbm, acc_out,
                      src_v, dstb0, dstb1, rows0, rows1, acc_sh,
                      dsem0, dsem1, gsem0, gsem1, ssem0, ssem1):
    dstb = (dstb0, dstb1)
    rows = (rows0, rows1)
    dsems = (dsem0, dsem1)
    gsems = (gsem0, gsem1)
    ssems = (ssem0, ssem1)
    c = lax.axis_index("c")
    s = lax.axis_index("s")
    wid = s * NC + c
    rs = s * ROWS_PER_TILE
    pltpu.sync_copy(zeros2_hbm, acc_sh.at[pl.ds(rs, ROWS_PER_TILE)])
    pltpu.sync_copy(src_hbm.at[wid], src_v)
    plsc.subcore_barrier()

    def _prefetch(i, b):
        pltpu.async_copy(dst_hbm.at[wid, i], dstb[b], dsems[b])
        pltpu.async_copy(hs_hbm.at[src_v.at[i]], rows[b], gsems[b])

    def _consume(i, b):
        pltpu.make_async_copy(dst_hbm.at[wid, i], dstb[b], dsems[b]).wait()
        pltpu.make_async_copy(hs_hbm.at[src_v.at[i]], rows[b],
                              gsems[b]).wait()
        pltpu.async_copy(rows[b], acc_sh.at[dstb[b]], ssems[b], add=True)

    def _swait(b):
        pltpu.make_async_copy(rows[b], acc_sh.at[dstb[b]], ssems[b]).wait()

    _prefetch(0, 0)
    _consume(0, 0)
    _prefetch(1, 1)

    def round_body(r, carry):
        i = 2 * r + 1
        _consume(i, 1)
        _swait(0)
        _prefetch(i + 1, 0)
        _consume(i + 1, 0)
        _swait(1)
        _prefetch(i + 2, 1)
        return carry

    lax.fori_loop(0, (CHUNKS_PER_TILE - 4) // 2, round_body, 0)
    _consume(CHUNKS_PER_TILE - 3, 1)
    _swait(0)
    _prefetch(CHUNKS_PER_TILE - 2, 0)
    _consume(CHUNKS_PER_TILE - 2, 0)
    _swait(1)
    _prefetch(CHUNKS_PER_TILE - 1, 1)
    _consume(CHUNKS_PER_TILE - 1, 1)
    _swait(0)
    _swait(1)
    plsc.subcore_barrier()
    pltpu.sync_copy(acc_sh.at[pl.ds(rs, ROWS_PER_TILE)],
                    acc_out.at[c, pl.ds(rs, ROWS_PER_TILE)])


def _mm1_body(deg_ref, x_ref, w_ref, hs_ref, dinv_ref):
    deg = deg_ref[0, :, 0:1] + deg_ref[1, :, 0:1] + 1.0
    dinv = lax.rsqrt(deg)
    h = jnp.dot(x_ref[...], w_ref[...], preferred_element_type=jnp.float32)
    hs_ref[...] = h * dinv
    dinv_ref[...] = jnp.broadcast_to(dinv, (RB, FEAT))


def _mid_body(acc_ref, hs_ref, dinv_ref, b_ref, w_ref, hs2_ref):
    dinv = dinv_ref[...]
    pre = (acc_ref[0] + acc_ref[1] + hs_ref[...]) * dinv + b_ref[...]
    o = jnp.maximum(pre, 0.0)
    hs2_ref[...] = jnp.dot(o, w_ref[...], preferred_element_type=jnp.float32) * dinv


def _final_body(acc_ref, hs_ref, dinv_ref, b_ref, out_ref):
    out_ref[...] = ((acc_ref[0] + acc_ref[1] + hs_ref[...]) * dinv_ref[...]
                    + b_ref[...])


def kernel(x, edge_index, W1, b1, W2, b2):
    src = edge_index[0].astype(jnp.int32)
    dst = edge_index[1].astype(jnp.int32)
    n_pad_e = EPAD - E
    pad_iota = jnp.arange(n_pad_e, dtype=jnp.int32)
    pad_src = pad_iota % N_NODES
    pad_dst = N_NODES + pad_iota % (NPAD - N_NODES)
    srcp = jnp.concatenate([src, pad_src]).reshape(NW, CHUNKS_PER_TILE, CHUNK)
    dstp = jnp.concatenate([dst, pad_dst]).reshape(NW, CHUNKS_PER_TILE, CHUNK)
    xp = jnp.pad(x, ((0, NPAD - N_NODES), (0, 0)))
    zeros1 = jnp.zeros((ROWS_PER_TILE, DEGW), jnp.float32)
    zeros2 = jnp.zeros((ROWS_PER_TILE, FEAT), jnp.float32)
    ones1 = jnp.ones((CHUNK, DEGW), jnp.float32)
    b1r = b1.reshape(1, FEAT)
    b2r = b2.reshape(1, FEAT)

    deg_p = _degree_kernel(dstp, ones1, zeros1)

    hs1, dinv = pl.pallas_call(
        _mm1_body,
        grid=(GRID,),
        in_specs=[
            pl.BlockSpec((NC, RB, DEGW), lambda i: (0, i, 0)),
            pl.BlockSpec((RB, FEAT), lambda i: (i, 0)),
            pl.BlockSpec((FEAT, FEAT), lambda i: (0, 0)),
        ],
        out_specs=[
            pl.BlockSpec((RB, FEAT), lambda i: (i, 0)),
            pl.BlockSpec((RB, FEAT), lambda i: (i, 0)),
        ],
        out_shape=[
            jax.ShapeDtypeStruct((NPAD, FEAT), jnp.float32),
            jax.ShapeDtypeStruct((NPAD, FEAT), jnp.float32),
        ],
    )(deg_p, xp, W1)

    acc1 = _aggregate_kernel(hs1, srcp, dstp, zeros2)

    hs2 = pl.pallas_call(
        _mid_body,
        grid=(GRID,),
        in_specs=[
            pl.BlockSpec((NC, RB, FEAT), lambda i: (0, i, 0)),
            pl.BlockSpec((RB, FEAT), lambda i: (i, 0)),
            pl.BlockSpec((RB, FEAT), lambda i: (i, 0)),
            pl.BlockSpec((1, FEAT), lambda i: (0, 0)),
            pl.BlockSpec((FEAT, FEAT), lambda i: (0, 0)),
        ],
        out_specs=pl.BlockSpec((RB, FEAT), lambda i: (i, 0)),
        out_shape=jax.ShapeDtypeStruct((NPAD, FEAT), jnp.float32),
    )(acc1, hs1, dinv, b1r, W2)

    acc2 = _aggregate_kernel(hs2, srcp, dstp, zeros2)

    out = pl.pallas_call(
        _final_body,
        grid=(GRID,),
        in_specs=[
            pl.BlockSpec((NC, RB, FEAT), lambda i: (0, i, 0)),
            pl.BlockSpec((RB, FEAT), lambda i: (i, 0)),
            pl.BlockSpec((RB, FEAT), lambda i: (i, 0)),
            pl.BlockSpec((1, FEAT), lambda i: (0, 0)),
        ],
        out_specs=pl.BlockSpec((RB, FEAT), lambda i: (i, 0)),
        out_shape=jax.ShapeDtypeStruct((NPAD, FEAT), jnp.float32),
    )(acc2, hs2, dinv, b2r)

    return out[:N_NODES]


# speedup vs baseline: 1.1234x; 1.1234x over previous
import functools

import jax
import jax.numpy as jnp
from jax import lax
from jax.experimental import pallas as pl
from jax.experimental.pallas import tpu as pltpu
from jax.experimental.pallas import tpu_sc as plsc

N_NODES = 10000
FEAT = 128
NPAD = 10240
TRASH = N_NODES
E = 320000
NC, NS = 2, 16
NW = NC * NS
EDGES_PER_TILE = E // NW
CHUNK = 128
CHUNKS_PER_TILE = 80
EPAD = NW * CHUNK * CHUNKS_PER_TILE
ROWS_PER_TILE = NPAD // NS
RB = 1024
GRID = NPAD // RB

_mesh = plsc.VectorSubcoreMesh(
    core_axis_name="c", subcore_axis_name="s", num_cores=NC, num_subcores=NS)


DEGW = 128


@functools.partial(
    pl.kernel,
    mesh=_mesh,
    out_type=jax.ShapeDtypeStruct((NC, NPAD, DEGW), jnp.float32),
    scratch_types=[
        pltpu.VMEM((CHUNKS_PER_TILE, CHUNK), jnp.int32),
        pltpu.VMEM((CHUNK, DEGW), jnp.float32),
        pltpu.VMEM_SHARED((NPAD, DEGW), jnp.float32),
        pltpu.SemaphoreType.DMA,
    ],
)
def _degree_kernel(dst_hbm, ones_hbm, zeros1_hbm, deg_out, dst_v, ones_v, deg_sh,
                   sem):
    c = lax.axis_index("c")
    s = lax.axis_index("s")
    wid = s * NC + c
    rs = s * ROWS_PER_TILE
    pltpu.sync_copy(zeros1_hbm, deg_sh.at[pl.ds(rs, ROWS_PER_TILE)])
    pltpu.sync_copy(ones_hbm, ones_v)
    pltpu.sync_copy(dst_hbm.at[wid], dst_v)
    plsc.subcore_barrier()

    win = 8
    for i in range(win):
        pltpu.async_copy(ones_v, deg_sh.at[dst_v.at[i]], sem, add=True)

    def body(i, carry):
        pltpu.make_async_copy(ones_v, deg_sh.at[dst_v.at[i]], sem).wait()
        pltpu.async_copy(ones_v, deg_sh.at[dst_v.at[i + win]], sem, add=True)
        return carry

    lax.fori_loop(0, CHUNKS_PER_TILE - win, body, 0)

    def drain(i, carry):
        pltpu.make_async_copy(ones_v, deg_sh.at[dst_v.at[i]], sem).wait()
        return carry

    lax.fori_loop(CHUNKS_PER_TILE - win, CHUNKS_PER_TILE, drain, 0)
    plsc.subcore_barrier()
    pltpu.sync_copy(deg_sh.at[pl.ds(rs, ROWS_PER_TILE)],
                    deg_out.at[c, pl.ds(rs, ROWS_PER_TILE)])


NBUF = 2
NROUNDS = CHUNKS_PER_TILE // NBUF


@functools.partial(
    pl.kernel,
    mesh=_mesh,
    out_type=jax.ShapeDtypeStruct((NC, NPAD, FEAT), jnp.float32),
    scratch_types=[
        pltpu.VMEM((CHUNKS_PER_TILE, CHUNK), jnp.int32),
        pltpu.VMEM((CHUNK,), jnp.int32),
        pltpu.VMEM((CHUNK,), jnp.int32),
        pltpu.VMEM((CHUNK, FEAT), jnp.float32),
        pltpu.VMEM((CHUNK, FEAT), jnp.float32),
        pltpu.VMEM_SHARED((NPAD, FEAT), jnp.float32),
        pltpu.SemaphoreType.DMA,
        pltpu.SemaphoreType.DMA,
        pltpu.SemaphoreType.DMA,
        pltpu.SemaphoreType.DMA,
    ],
)
def _aggregate_kernel(hs_hbm, src_hbm, dst_hbm, zeros2_hbm, acc_out,
                      src_v, dstb0, dstb1, rows0, rows1, acc_sh,
                      dsem0, dsem1, gsem0, gsem1):
    dstb = (dstb0, dstb1)
    rows = (rows0, rows1)
    dsems = (dsem0, dsem1)
    gsems = (gsem0, gsem1)
    c = lax.axis_index("c")
    s = lax.axis_index("s")
    wid = s * NC + c
    rs = s * ROWS_PER_TILE
    pltpu.sync_copy(zeros2_hbm, acc_sh.at[pl.ds(rs, ROWS_PER_TILE)])
    pltpu.sync_copy(src_hbm.at[wid], src_v)
    plsc.subcore_barrier()

    for b in range(NBUF):
        pltpu.async_copy(dst_hbm.at[wid, b], dstb[b], dsems[b])
        pltpu.async_copy(hs_hbm.at[src_v.at[b]], rows[b], gsems[b])

    def round_body(g, carry):
        for b in range(NBUF):
            i = g * NBUF + b
            pltpu.make_async_copy(dst_hbm.at[wid, i], dstb[b],
                                  dsems[b]).wait()
            pltpu.make_async_copy(hs_hbm.at[src_v.at[i]], rows[b],
                                  gsems[b]).wait()
            pltpu.sync_copy(rows[b], acc_sh.at[dstb[b]], add=True)
            pltpu.async_copy(dst_hbm.at[wid, i + NBUF], dstb[b], dsems[b])
            pltpu.async_copy(hs_hbm.at[src_v.at[i + NBUF]], rows[b],
                             gsems[b])
        return carry

    lax.fori_loop(0, NROUNDS - 1, round_body, 0)
    for b in range(NBUF):
        i = (NROUNDS - 1) * NBUF + b
        pltpu.make_async_copy(dst_hbm.at[wid, i], dstb[b], dsems[b]).wait()
        pltpu.make_async_copy(hs_hbm.at[src_v.at[i]], rows[b],
                              gsems[b]).wait()
        pltpu.sync_copy(rows[b], acc_sh.at[dstb[b]], add=True)
    plsc.subcore_barrier()
    pltpu.sync_copy(acc_sh.at[pl.ds(rs, ROWS_PER_TILE)],
                    acc_out.at[c, pl.ds(rs, ROWS_PER_TILE)])


def _mm1_body(deg_ref, x_ref, w_ref, hs_ref, dinv_ref):
    deg = deg_ref[0, :, 0:1] + deg_ref[1, :, 0:1] + 1.0
    dinv = lax.rsqrt(deg)
    h = jnp.dot(x_ref[...], w_ref[...], preferred_element_type=jnp.float32)
    hs_ref[...] = h * dinv
    dinv_ref[...] = jnp.broadcast_to(dinv, (RB, FEAT))


def _mid_body(acc_ref, hs_ref, dinv_ref, b_ref, w_ref, hs2_ref):
    dinv = dinv_ref[...]
    pre = (acc_ref[0] + acc_ref[1] + hs_ref[...]) * dinv + b_ref[...]
    o = jnp.maximum(pre, 0.0)
    hs2_ref[...] = jnp.dot(o, w_ref[...], preferred_element_type=jnp.float32) * dinv


def _final_body(acc_ref, hs_ref, dinv_ref, b_ref, out_ref):
    out_ref[...] = ((acc_ref[0] + acc_ref[1] + hs_ref[...]) * dinv_ref[...]
                    + b_ref[...])


def kernel(x, edge_index, W1, b1, W2, b2):
    src = edge_index[0].astype(jnp.int32)
    dst = edge_index[1].astype(jnp.int32)
    n_pad_e = EPAD - E
    pad_iota = jnp.arange(n_pad_e, dtype=jnp.int32)
    pad_src = pad_iota % N_NODES
    pad_dst = N_NODES + pad_iota % (NPAD - N_NODES)
    srcp = jnp.concatenate([src, pad_src]).reshape(NW, CHUNKS_PER_TILE, CHUNK)
    dstp = jnp.concatenate([dst, pad_dst]).reshape(NW, CHUNKS_PER_TILE, CHUNK)
    xp = jnp.pad(x, ((0, NPAD - N_NODES), (0, 0)))
    zeros1 = jnp.zeros((ROWS_PER_TILE, DEGW), jnp.float32)
    zeros2 = jnp.zeros((ROWS_PER_TILE, FEAT), jnp.float32)
    ones1 = jnp.ones((CHUNK, DEGW), jnp.float32)
    b1r = b1.reshape(1, FEAT)
    b2r = b2.reshape(1, FEAT)

    deg_p = _degree_kernel(dstp, ones1, zeros1)

    hs1, dinv = pl.pallas_call(
        _mm1_body,
        grid=(GRID,),
        in_specs=[
            pl.BlockSpec((NC, RB, DEGW), lambda i: (0, i, 0)),
            pl.BlockSpec((RB, FEAT), lambda i: (i, 0)),
            pl.BlockSpec((FEAT, FEAT), lambda i: (0, 0)),
        ],
        out_specs=[
            pl.BlockSpec((RB, FEAT), lambda i: (i, 0)),
            pl.BlockSpec((RB, FEAT), lambda i: (i, 0)),
        ],
        out_shape=[
            jax.ShapeDtypeStruct((NPAD, FEAT), jnp.float32),
            jax.ShapeDtypeStruct((NPAD, FEAT), jnp.float32),
        ],
    )(deg_p, xp, W1)

    acc1 = _aggregate_kernel(hs1, srcp, dstp, zeros2)

    hs2 = pl.pallas_call(
        _mid_body,
        grid=(GRID,),
        in_specs=[
            pl.BlockSpec((NC, RB, FEAT), lambda i: (0, i, 0)),
            pl.BlockSpec((RB, FEAT), lambda i: (i, 0)),
            pl.BlockSpec((RB, FEAT), lambda i: (i, 0)),
            pl.BlockSpec((1, FEAT), lambda i: (0, 0)),
            pl.BlockSpec((FEAT, FEAT), lambda i: (0, 0)),
        ],
        out_specs=pl.BlockSpec((RB, FEAT), lambda i: (i, 0)),
        out_shape=jax.ShapeDtypeStruct((NPAD, FEAT), jnp.float32),
    )(acc1, hs1, dinv, b1r, W2)

    acc2 = _aggregate_kernel(hs2, srcp, dstp, zeros2)

    out = pl.pallas_call(
        _final_body,
        grid=(GRID,),
        in_specs=[
            pl.BlockSpec((NC, RB, FEAT), lambda i: (0, i, 0)),
            pl.BlockSpec((RB, FEAT), lambda i: (i, 0)),
            pl.BlockSpec((RB, FEAT), lambda i: (i, 0)),
            pl.BlockSpec((1, FEAT), lambda i: (0, 0)),
        ],
        out_specs=pl.BlockSpec((RB, FEAT), lambda i: (i, 0)),
        out_shape=jax.ShapeDtypeStruct((NPAD, FEAT), jnp.float32),
    )(acc2, hs2, dinv, b2r)

    return out[:N_NODES]


# speedup vs baseline: 1.1287x; 1.0047x over previous
import functools

import jax
import jax.numpy as jnp
from jax import lax
from jax.experimental import pallas as pl
from jax.experimental.pallas import tpu as pltpu
from jax.experimental.pallas import tpu_sc as plsc

N_NODES = 10000
FEAT = 128
NPAD = 10240
TRASH = N_NODES
E = 320000
NC, NS = 2, 16
NW = NC * NS
EDGES_PER_TILE = E // NW
CHUNK = 128
CHUNKS_PER_TILE = 80
EPAD = NW * CHUNK * CHUNKS_PER_TILE
ROWS_PER_TILE = NPAD // NS
RB = 1024
GRID = NPAD // RB

_mesh = plsc.VectorSubcoreMesh(
    core_axis_name="c", subcore_axis_name="s", num_cores=NC, num_subcores=NS)


DEGW = 128


@functools.partial(
    pl.kernel,
    mesh=_mesh,
    out_type=jax.ShapeDtypeStruct((NC, NPAD, DEGW), jnp.float32),
    scratch_types=[
        pltpu.VMEM((CHUNKS_PER_TILE, CHUNK), jnp.int32),
        pltpu.VMEM((CHUNK, DEGW), jnp.float32),
        pltpu.VMEM_SHARED((NPAD, DEGW), jnp.float32),
        pltpu.SemaphoreType.DMA,
    ],
)
def _degree_kernel(dst_hbm, ones_hbm, zeros1_hbm, deg_out, dst_v, ones_v, deg_sh,
                   sem):
    c = lax.axis_index("c")
    s = lax.axis_index("s")
    wid = s * NC + c
    rs = s * ROWS_PER_TILE
    pltpu.sync_copy(zeros1_hbm, deg_sh.at[pl.ds(rs, ROWS_PER_TILE)])
    pltpu.sync_copy(ones_hbm, ones_v)
    pltpu.sync_copy(dst_hbm.at[wid], dst_v)
    plsc.subcore_barrier()

    win = 8
    for i in range(win):
        pltpu.async_copy(ones_v, deg_sh.at[dst_v.at[i]], sem, add=True)

    def body(i, carry):
        pltpu.make_async_copy(ones_v, deg_sh.at[dst_v.at[i]], sem).wait()
        pltpu.async_copy(ones_v, deg_sh.at[dst_v.at[i + win]], sem, add=True)
        return carry

    lax.fori_loop(0, CHUNKS_PER_TILE - win, body, 0)

    def drain(i, carry):
        pltpu.make_async_copy(ones_v, deg_sh.at[dst_v.at[i]], sem).wait()
        return carry

    lax.fori_loop(CHUNKS_PER_TILE - win, CHUNKS_PER_TILE, drain, 0)
    plsc.subcore_barrier()
    pltpu.sync_copy(deg_sh.at[pl.ds(rs, ROWS_PER_TILE)],
                    deg_out.at[c, pl.ds(rs, ROWS_PER_TILE)])


NBUF = 2
NROUNDS = CHUNKS_PER_TILE // NBUF


@functools.partial(
    pl.kernel,
    mesh=_mesh,
    out_type=jax.ShapeDtypeStruct((NC, NPAD, FEAT), jnp.float32),
    scratch_types=[
        pltpu.VMEM((CHUNKS_PER_TILE, CHUNK), jnp.int32),
        pltpu.VMEM((CHUNK,), jnp.int32),
        pltpu.VMEM((CHUNK,), jnp.int32),
        pltpu.VMEM((CHUNK, FEAT), jnp.float32),
        pltpu.VMEM((CHUNK, FEAT), jnp.float32),
        pltpu.VMEM_SHARED((NPAD, FEAT), jnp.float32),
        pltpu.SemaphoreType.DMA,
        pltpu.SemaphoreType.DMA,
        pltpu.SemaphoreType.DMA,
        pltpu.SemaphoreType.DMA,
    ],
)
def _aggregate_kernel(hs_hbm, src_hbm, dst_hbm, zeros2_hbm, acc_out,
                      src_v, dstb0, dstb1, rows0, rows1, acc_sh,
                      dsem0, dsem1, gsem0, gsem1):
    dstb = (dstb0, dstb1)
    rows = (rows0, rows1)
    dsems = (dsem0, dsem1)
    gsems = (gsem0, gsem1)
    c = lax.axis_index("c")
    s = lax.axis_index("s")
    wid = s * NC + c
    rs = s * ROWS_PER_TILE
    pltpu.sync_copy(zeros2_hbm, acc_sh.at[pl.ds(rs, ROWS_PER_TILE)])
    pltpu.sync_copy(src_hbm.at[wid], src_v)
    plsc.subcore_barrier()

    for b in range(NBUF):
        pltpu.async_copy(dst_hbm.at[wid, b], dstb[b], dsems[b])
        pltpu.async_copy(hs_hbm.at[src_v.at[b]], rows[b], gsems[b])

    def round_body(g, carry):
        for b in range(NBUF):
            i = g * NBUF + b
            pltpu.make_async_copy(dst_hbm.at[wid, i], dstb[b],
                                  dsems[b]).wait()
            pltpu.make_async_copy(hs_hbm.at[src_v.at[i]], rows[b],
                                  gsems[b]).wait()
            pltpu.sync_copy(rows[b], acc_sh.at[dstb[b]], add=True)
            pltpu.async_copy(dst_hbm.at[wid, i + NBUF], dstb[b], dsems[b])
            pltpu.async_copy(hs_hbm.at[src_v.at[i + NBUF]], rows[b],
                             gsems[b])
        return carry

    lax.fori_loop(0, NROUNDS - 1, round_body, 0)
    for b in range(NBUF):
        i = (NROUNDS - 1) * NBUF + b
        pltpu.make_async_copy(dst_hbm.at[wid, i], dstb[b], dsems[b]).wait()
        pltpu.make_async_copy(hs_hbm.at[src_v.at[i]], rows[b],
                              gsems[b]).wait()
        pltpu.sync_copy(rows[b], acc_sh.at[dstb[b]], add=True)
    plsc.subcore_barrier()
    pltpu.sync_copy(acc_sh.at[pl.ds(rs, ROWS_PER_TILE)],
                    acc_out.at[c, pl.ds(rs, ROWS_PER_TILE)])


DINVW = 8


def _mm0_body(x_ref, w_ref, h_ref):
    h_ref[...] = jnp.dot(x_ref[...], w_ref[...],
                         preferred_element_type=jnp.float32)


def _scale_body(deg_ref, h_ref, hs_ref, dinv_ref):
    deg = deg_ref[0, :, 0:1] + deg_ref[1, :, 0:1] + 1.0
    dinv = lax.rsqrt(deg)
    hs_ref[...] = h_ref[...] * dinv
    dinv_ref[...] = jnp.broadcast_to(dinv, (RB, DINVW))


def _mid_body(acc_ref, hs_ref, dinv_ref, b_ref, w_ref, hs2_ref):
    dinv = dinv_ref[:, 0:1]
    pre = (acc_ref[0] + acc_ref[1] + hs_ref[...]) * dinv + b_ref[...]
    o = jnp.maximum(pre, 0.0)
    hs2_ref[...] = jnp.dot(o, w_ref[...], preferred_element_type=jnp.float32) * dinv


def _final_body(acc_ref, hs_ref, dinv_ref, b_ref, out_ref):
    out_ref[...] = ((acc_ref[0] + acc_ref[1] + hs_ref[...]) * dinv_ref[:, 0:1]
                    + b_ref[...])


def kernel(x, edge_index, W1, b1, W2, b2):
    src = edge_index[0].astype(jnp.int32)
    dst = edge_index[1].astype(jnp.int32)
    n_pad_e = EPAD - E
    pad_iota = jnp.arange(n_pad_e, dtype=jnp.int32)
    pad_src = pad_iota % N_NODES
    pad_dst = N_NODES + pad_iota % (NPAD - N_NODES)
    srcp = jnp.concatenate([src, pad_src]).reshape(NW, CHUNKS_PER_TILE, CHUNK)
    dstp = jnp.concatenate([dst, pad_dst]).reshape(NW, CHUNKS_PER_TILE, CHUNK)
    xp = jnp.pad(x, ((0, NPAD - N_NODES), (0, 0)))
    zeros1 = jnp.zeros((ROWS_PER_TILE, DEGW), jnp.float32)
    zeros2 = jnp.zeros((ROWS_PER_TILE, FEAT), jnp.float32)
    ones1 = jnp.ones((CHUNK, DEGW), jnp.float32)
    b1r = b1.reshape(1, FEAT)
    b2r = b2.reshape(1, FEAT)

    deg_p = _degree_kernel(dstp, ones1, zeros1)

    h1 = pl.pallas_call(
        _mm0_body,
        grid=(GRID,),
        in_specs=[
            pl.BlockSpec((RB, FEAT), lambda i: (i, 0)),
            pl.BlockSpec((FEAT, FEAT), lambda i: (0, 0)),
        ],
        out_specs=pl.BlockSpec((RB, FEAT), lambda i: (i, 0)),
        out_shape=jax.ShapeDtypeStruct((NPAD, FEAT), jnp.float32),
    )(xp, W1)

    hs1, dinv = pl.pallas_call(
        _scale_body,
        grid=(GRID,),
        in_specs=[
            pl.BlockSpec((NC, RB, DEGW), lambda i: (0, i, 0)),
            pl.BlockSpec((RB, FEAT), lambda i: (i, 0)),
        ],
        out_specs=[
            pl.BlockSpec((RB, FEAT), lambda i: (i, 0)),
            pl.BlockSpec((RB, DINVW), lambda i: (i, 0)),
        ],
        out_shape=[
            jax.ShapeDtypeStruct((NPAD, FEAT), jnp.float32),
            jax.ShapeDtypeStruct((NPAD, DINVW), jnp.float32),
        ],
    )(deg_p, h1)

    acc1 = _aggregate_kernel(hs1, srcp, dstp, zeros2)

    hs2 = pl.pallas_call(
        _mid_body,
        grid=(GRID,),
        in_specs=[
            pl.BlockSpec((NC, RB, FEAT), lambda i: (0, i, 0)),
            pl.BlockSpec((RB, FEAT), lambda i: (i, 0)),
            pl.BlockSpec((RB, DINVW), lambda i: (i, 0)),
            pl.BlockSpec((1, FEAT), lambda i: (0, 0)),
            pl.BlockSpec((FEAT, FEAT), lambda i: (0, 0)),
        ],
        out_specs=pl.BlockSpec((RB, FEAT), lambda i: (i, 0)),
        out_shape=jax.ShapeDtypeStruct((NPAD, FEAT), jnp.float32),
    )(acc1, hs1, dinv, b1r, W2)

    acc2 = _aggregate_kernel(hs2, srcp, dstp, zeros2)

    RBF = 1000
    out = pl.pallas_call(
        _final_body,
        grid=(N_NODES // RBF,),
        in_specs=[
            pl.BlockSpec((NC, RBF, FEAT), lambda i: (0, i, 0)),
            pl.BlockSpec((RBF, FEAT), lambda i: (i, 0)),
            pl.BlockSpec((RBF, DINVW), lambda i: (i, 0)),
            pl.BlockSpec((1, FEAT), lambda i: (0, 0)),
        ],
        out_specs=pl.BlockSpec((RBF, FEAT), lambda i: (i, 0)),
        out_shape=jax.ShapeDtypeStruct((N_NODES, FEAT), jnp.float32),
    )(acc2, hs2, dinv, b2r)

    return out
